# baseline (speedup 1.0000x reference)
import jax
import jax.numpy as jnp
from jax.experimental import pallas as pl
from jax.experimental.pallas import tpu as pltpu

_EPS = 1e-7
_LN2 = 0.6931471805599453
_ROWS = 8
_NSTREAM = 4
_FOLD = 16

_MANT_MASK = 0x007FFFFF
_ONE_BITS = 0x3F800000


def _nll_body(p0, p1, p2, p3, o_hbm, out_ref, accm_ref, acce_ref,
              obuf_ref, sem):
    i = pl.program_id(0)
    nsteps = pl.num_programs(0)
    rows = _ROWS * _NSTREAM

    cp = pltpu.make_async_copy(
        o_hbm.at[pl.ds(i * rows, rows)], obuf_ref, sem)
    cp.start()
    cp.wait()

    o = obuf_ref[...]
    one = jnp.float32(1.0)
    q0 = jnp.where(o[0:_ROWS] == 1, p0[...] + _EPS, one)
    q1 = jnp.where(o[_ROWS:2 * _ROWS] == 1, p1[...] + _EPS, one)
    q2 = jnp.where(o[2 * _ROWS:3 * _ROWS] == 1, p2[...] + _EPS, one)
    q3 = jnp.where(o[3 * _ROWS:4 * _ROWS] == 1, p3[...] + _EPS, one)
    P = (q0 * q1) * (q2 * q3)

    @pl.when(i == 0)
    def _first():
        b = P.view(jnp.int32)
        acce_ref[...] = b >> 23
        accm_ref[...] = ((b & _MANT_MASK) | _ONE_BITS).view(jnp.float32)

    @pl.when(i > 0)
    def _fold():
        t = accm_ref[...] * P
        b = t.view(jnp.int32)
        acce_ref[...] += b >> 23
        accm_ref[...] = ((b & _MANT_MASK) | _ONE_BITS).view(jnp.float32)

    @pl.when(i == nsteps - 1)
    def _finish():
        am = accm_ref[...]
        ae = acce_ref[...]
        R2, C = am.shape
        fw = C // _FOLD
        fm = am[:, 0:fw]
        fe = jnp.zeros((R2, fw), jnp.int32)
        for k in range(1, _FOLD):
            t = fm * am[:, k * fw:(k + 1) * fw]
            b = t.view(jnp.int32)
            fe += b >> 23
            fm = ((b & _MANT_MASK) | _ONE_BITS).view(jnp.float32)
        n_renorm = nsteps * R2 * C + (_FOLD - 1) * R2 * fw
        e_total = jnp.sum(ae) + jnp.sum(fe) - jnp.int32(127 * n_renorm)
        ln_total = (jnp.float32(_LN2) * e_total.astype(jnp.float32)
                    + jnp.sum(jnp.log(fm)))
        out_ref[0, 0] = -ln_total


def kernel(predicted, observed):
    B, N = predicted.shape
    rows_per_step = _ROWS * _NSTREAM
    nsteps = B // rows_per_step
    grid = (nsteps,)

    def p_spec(s):
        return pl.BlockSpec((_ROWS, N), lambda i, s=s: (i * _NSTREAM + s, 0))

    out = pl.pallas_call(
        _nll_body,
        grid=grid,
        in_specs=[p_spec(0), p_spec(1), p_spec(2), p_spec(3),
                  pl.BlockSpec(memory_space=pl.ANY)],
        out_specs=pl.BlockSpec(memory_space=pltpu.SMEM),
        out_shape=jax.ShapeDtypeStruct((1, 1), jnp.float32),
        scratch_shapes=[
            pltpu.VMEM((_ROWS, N), jnp.float32),
            pltpu.VMEM((_ROWS, N), jnp.int32),
            pltpu.VMEM((rows_per_step, N), jnp.int8),
            pltpu.SemaphoreType.DMA,
        ],
    )(predicted, predicted, predicted, predicted, observed.view(jnp.int8))
    return out[0, 0] / B
